# baseline (speedup 1.0000x reference)
import functools

import jax
import jax.numpy as jnp
from jax import lax
from jax.experimental import pallas as pl
from jax.experimental.pallas import tpu as pltpu
from jax.experimental.pallas import tpu_sc as plsc

N = 10000
D0 = 128
H0 = 128
C1 = 64
ND0 = 5000
ND1 = 1000
E0 = 320000
E1 = 64000

NC = 2
NS = 16
NW = NC * NS
L = 16

PAD0 = 5120
PAD1 = 1024


def _make_sc_agg(EP, D, PAD, CH):
    NB = 8
    EW = EP // NW
    CHUNKS = EW // CH
    GROUPS = CHUNKS // NB
    TAIL = CHUNKS % NB
    RPT = PAD // NS
    mesh = plsc.VectorSubcoreMesh(core_axis_name="c", subcore_axis_name="s")

    @functools.partial(
        pl.kernel,
        mesh=mesh,
        out_type=(
            jax.ShapeDtypeStruct((NC, PAD, D), jnp.float32),
            jax.ShapeDtypeStruct((NW, PAD), jnp.float32),
        ),
        scratch_types=[
            [pltpu.VMEM((CH,), jnp.int32)] * NB,
            [pltpu.VMEM((CH,), jnp.int32)] * NB,
            [pltpu.VMEM((CH, D), jnp.float32)] * NB,
            pltpu.VMEM((PAD,), jnp.float32),
            pltpu.VMEM_SHARED((PAD, D), jnp.float32),
            [pltpu.SemaphoreType.DMA] * NB,
        ],
        compiler_params=pltpu.CompilerParams(needs_layout_passes=False),
    )
    def agg_kernel(table, src_e, dst_e, zacc, zdeg, agg_out, deg_out,
                   src_v, dst_v, rows_v, deg_v, sh_acc, sems):
        cid = lax.axis_index("c")
        sid = lax.axis_index("s")
        wid = sid * NC + cid
        pltpu.sync_copy(zacc.at[pl.ds(sid * RPT, RPT)],
                        sh_acc.at[pl.ds(sid * RPT, RPT)])
        pltpu.sync_copy(zdeg, deg_v)
        plsc.subcore_barrier()

        base = wid * EW
        ones = jnp.full((L,), 1.0, jnp.float32)

        def start(off, b):
            pltpu.sync_copy(src_e.at[pl.ds(off, CH)], src_v[b])
            pltpu.sync_copy(dst_e.at[pl.ds(off, CH)], dst_v[b])
            return pltpu.async_copy(table.at[src_v[b]], rows_v[b], sems[b])

        def finish(handle, b):
            handle.wait()
            pltpu.sync_copy(rows_v[b], sh_acc.at[dst_v[b]], add=True)
            for j in range(CH // L):
                dv = dst_v[b][pl.ds(j * L, L)]
                plsc.addupdate_scatter(deg_v, [dv], ones)

        def outer(o, carry):
            off = base + o * (NB * CH)
            handles = [start(off + b * CH, b) for b in range(NB)]
            for b in range(NB):
                finish(handles[b], b)
            return carry

        lax.fori_loop(0, GROUPS, outer, 0)
        for t in range(TAIL):
            off = base + (GROUPS * NB + t) * CH
            finish(start(off, t), t)
        plsc.subcore_barrier()
        pltpu.sync_copy(sh_acc.at[pl.ds(sid * RPT, RPT)],
                        agg_out.at[cid, pl.ds(sid * RPT, RPT)])
        pltpu.sync_copy(deg_v, deg_out.at[wid])

    return agg_kernel


CH = 80
E0P = E0
E1P = E1

_sc_agg0 = _make_sc_agg(E0P, D0, PAD0, CH)
_sc_agg1 = _make_sc_agg(E1P, H0, PAD1, CH)


def _pad_edges(ei, ep, nd, pad):
    npad = ep - ei.shape[1]
    src = jnp.concatenate([ei[0], jnp.zeros((npad,), jnp.int32)])
    cyc = nd + jnp.arange(npad, dtype=jnp.int32) % jnp.int32(pad - nd)
    dst = jnp.concatenate([ei[1], cyc])
    return src, dst


def _tc_layer0_body(agg_ref, deg_ref, h_ref, wn0, ws0, b0r, g0r, be0r,
                    ws1, b1r, x_ref, xs_ref):
    deg = jnp.sum(deg_ref[...], axis=0)[:ND0]
    aggs = agg_ref[0, :ND0, :] + agg_ref[1, :ND0, :]
    mean = aggs / jnp.maximum(deg, 1.0)[:, None]
    dn = (((1,), (1,)), ((), ()))
    z = (lax.dot_general(mean, wn0[...], dn, preferred_element_type=jnp.float32)
         + lax.dot_general(h_ref[...], ws0[...], dn, preferred_element_type=jnp.float32)
         + b0r[...][None, :])
    mu = jnp.mean(z, axis=-1, keepdims=True)
    zc = z - mu
    var = jnp.mean(zc * zc, axis=-1, keepdims=True)
    xn = zc * lax.rsqrt(var + 1e-5)
    x = jnp.maximum(xn * g0r[...][None, :] + be0r[...][None, :], 0.0)
    x_ref[...] = x
    x1 = x[:ND1]
    xs_ref[...] = (lax.dot_general(x1, ws1[...], dn, preferred_element_type=jnp.float32)
                   + b1r[...][None, :])


def _tc_layer1_body(agg_ref, deg_ref, xs_ref, wn1, out_ref):
    deg = jnp.sum(deg_ref[...], axis=0)[:ND1]
    aggs = agg_ref[0, :ND1, :] + agg_ref[1, :ND1, :]
    mean = aggs / jnp.maximum(deg, 1.0)[:, None]
    dn = (((1,), (1,)), ((), ()))
    out_ref[...] = (lax.dot_general(mean, wn1[...], dn,
                                    preferred_element_type=jnp.float32)
                    + xs_ref[...])


def kernel(h, edge_index0, edge_index1, W_neigh0, W_self0, b0,
           gamma0, beta0, W_neigh1, W_self1, b1):
    zacc0 = jnp.zeros((PAD0, D0), jnp.float32)
    zdeg0 = jnp.zeros((PAD0,), jnp.float32)
    zacc1 = jnp.zeros((PAD1, H0), jnp.float32)
    zdeg1 = jnp.zeros((PAD1,), jnp.float32)

    src0, dst0 = _pad_edges(edge_index0, E0P, ND0, PAD0)
    agg0, deg0 = _sc_agg0(h, src0, dst0, zacc0, zdeg0)

    h5k = lax.slice(h, (0, 0), (ND0, D0))
    x, xs = pl.pallas_call(
        _tc_layer0_body,
        out_shape=(
            jax.ShapeDtypeStruct((ND0, H0), jnp.float32),
            jax.ShapeDtypeStruct((ND1, C1), jnp.float32),
        ),
    )(agg0, deg0, h5k, W_neigh0, W_self0, b0, gamma0, beta0,
      W_self1, b1)

    src1, dst1 = _pad_edges(edge_index1, E1P, ND1, PAD1)
    agg1, deg1 = _sc_agg1(x, src1, dst1, zacc1, zdeg1)

    out = pl.pallas_call(
        _tc_layer1_body,
        out_shape=jax.ShapeDtypeStruct((ND1, C1), jnp.float32),
    )(agg1, deg1, xs, W_neigh1)
    return out


# speedup vs baseline: 1.0021x; 1.0021x over previous
import functools

import jax
import jax.numpy as jnp
from jax import lax
from jax.experimental import pallas as pl
from jax.experimental.pallas import tpu as pltpu
from jax.experimental.pallas import tpu_sc as plsc

N = 10000
D0 = 128
H0 = 128
C1 = 64
ND0 = 5000
ND1 = 1000
E0 = 320000
E1 = 64000

NC = 2
NS = 16
NW = NC * NS
L = 16

PAD0 = 5120
PAD1 = 1024


def _make_sc_agg(EP, D, PAD, CH, TROWS):
    NB = 4
    EW = EP // NW
    CHUNKS = EW // CH
    GROUPS = CHUNKS // NB
    TAIL = CHUNKS % NB
    RPT = PAD // NS
    mesh = plsc.VectorSubcoreMesh(core_axis_name="c", subcore_axis_name="s")

    @functools.partial(
        pl.kernel,
        mesh=mesh,
        out_type=(
            jax.ShapeDtypeStruct((NC, PAD, D), jnp.float32),
            jax.ShapeDtypeStruct((NW, PAD), jnp.float32),
        ),
        scratch_types=[
            [pltpu.VMEM((CH,), jnp.int32)] * NB,
            [pltpu.VMEM((CH,), jnp.int32)] * NB,
            [pltpu.VMEM((CH, D), jnp.float32)] * NB,
            pltpu.VMEM((PAD,), jnp.float32),
            pltpu.VMEM_SHARED((PAD, D), jnp.float32),
            pltpu.VMEM_SHARED((TROWS, D), jnp.float32),
            [pltpu.SemaphoreType.DMA] * NB,
        ],
        compiler_params=pltpu.CompilerParams(needs_layout_passes=False),
    )
    def agg_kernel(table, src_e, dst_e, zacc, zdeg, agg_out, deg_out,
                   src_v, dst_v, rows_v, deg_v, sh_acc, sh_tab, sems):
        cid = lax.axis_index("c")
        sid = lax.axis_index("s")
        wid = sid * NC + cid
        TPR = TROWS // NS
        pltpu.sync_copy(table.at[pl.ds(sid * TPR, TPR)],
                        sh_tab.at[pl.ds(sid * TPR, TPR)])
        pltpu.sync_copy(zacc.at[pl.ds(sid * RPT, RPT)],
                        sh_acc.at[pl.ds(sid * RPT, RPT)])
        pltpu.sync_copy(zdeg, deg_v)
        plsc.subcore_barrier()

        base = wid * EW
        ones = jnp.full((L,), 1.0, jnp.float32)

        def start(off, b):
            pltpu.sync_copy(src_e.at[pl.ds(off, CH)], src_v[b])
            pltpu.sync_copy(dst_e.at[pl.ds(off, CH)], dst_v[b])
            return pltpu.async_copy(sh_tab.at[src_v[b]], rows_v[b], sems[b])

        def finish(handle, b):
            handle.wait()
            pltpu.sync_copy(rows_v[b], sh_acc.at[dst_v[b]], add=True)
            for j in range(CH // L):
                dv = dst_v[b][pl.ds(j * L, L)]
                plsc.addupdate_scatter(deg_v, [dv], ones)

        def outer(o, carry):
            off = base + o * (NB * CH)
            handles = [start(off + b * CH, b) for b in range(NB)]
            for b in range(NB):
                finish(handles[b], b)
            return carry

        lax.fori_loop(0, GROUPS, outer, 0)
        for t in range(TAIL):
            off = base + (GROUPS * NB + t) * CH
            finish(start(off, t), t)
        plsc.subcore_barrier()
        pltpu.sync_copy(sh_acc.at[pl.ds(sid * RPT, RPT)],
                        agg_out.at[cid, pl.ds(sid * RPT, RPT)])
        pltpu.sync_copy(deg_v, deg_out.at[wid])

    return agg_kernel


CH = 80
E0P = E0
E1P = E1

_sc_agg0 = _make_sc_agg(E0P, D0, PAD0, CH, PAD0)
_sc_agg1 = _make_sc_agg(E1P, H0, PAD1, CH, PAD1)


def _pad_edges(ei, ep, nd, pad):
    npad = ep - ei.shape[1]
    src = jnp.concatenate([ei[0], jnp.zeros((npad,), jnp.int32)])
    cyc = nd + jnp.arange(npad, dtype=jnp.int32) % jnp.int32(pad - nd)
    dst = jnp.concatenate([ei[1], cyc])
    return src, dst


def _tc_layer0_body(agg_ref, deg_ref, h_ref, wn0, ws0, b0r, g0r, be0r,
                    ws1, b1r, x_ref, xs_ref):
    deg = jnp.sum(deg_ref[...], axis=0)[:ND0]
    aggs = agg_ref[0, :ND0, :] + agg_ref[1, :ND0, :]
    mean = aggs / jnp.maximum(deg, 1.0)[:, None]
    dn = (((1,), (1,)), ((), ()))
    z = (lax.dot_general(mean, wn0[...], dn, preferred_element_type=jnp.float32)
         + lax.dot_general(h_ref[...], ws0[...], dn, preferred_element_type=jnp.float32)
         + b0r[...][None, :])
    mu = jnp.mean(z, axis=-1, keepdims=True)
    zc = z - mu
    var = jnp.mean(zc * zc, axis=-1, keepdims=True)
    xn = zc * lax.rsqrt(var + 1e-5)
    x = jnp.maximum(xn * g0r[...][None, :] + be0r[...][None, :], 0.0)
    x_ref[...] = x
    x1 = x[:ND1]
    xs_ref[...] = (lax.dot_general(x1, ws1[...], dn, preferred_element_type=jnp.float32)
                   + b1r[...][None, :])


def _tc_layer1_body(agg_ref, deg_ref, xs_ref, wn1, out_ref):
    deg = jnp.sum(deg_ref[...], axis=0)[:ND1]
    aggs = agg_ref[0, :ND1, :] + agg_ref[1, :ND1, :]
    mean = aggs / jnp.maximum(deg, 1.0)[:, None]
    dn = (((1,), (1,)), ((), ()))
    out_ref[...] = (lax.dot_general(mean, wn1[...], dn,
                                    preferred_element_type=jnp.float32)
                    + xs_ref[...])


def kernel(h, edge_index0, edge_index1, W_neigh0, W_self0, b0,
           gamma0, beta0, W_neigh1, W_self1, b1):
    zacc0 = jnp.zeros((PAD0, D0), jnp.float32)
    zdeg0 = jnp.zeros((PAD0,), jnp.float32)
    zacc1 = jnp.zeros((PAD1, H0), jnp.float32)
    zdeg1 = jnp.zeros((PAD1,), jnp.float32)

    src0, dst0 = _pad_edges(edge_index0, E0P, ND0, PAD0)
    agg0, deg0 = _sc_agg0(h, src0, dst0, zacc0, zdeg0)

    h5k = lax.slice(h, (0, 0), (ND0, D0))
    x, xs = pl.pallas_call(
        _tc_layer0_body,
        out_shape=(
            jax.ShapeDtypeStruct((ND0, H0), jnp.float32),
            jax.ShapeDtypeStruct((ND1, C1), jnp.float32),
        ),
    )(agg0, deg0, h5k, W_neigh0, W_self0, b0, gamma0, beta0,
      W_self1, b1)

    src1, dst1 = _pad_edges(edge_index1, E1P, ND1, PAD1)
    agg1, deg1 = _sc_agg1(x, src1, dst1, zacc1, zdeg1)

    out = pl.pallas_call(
        _tc_layer1_body,
        out_shape=jax.ShapeDtypeStruct((ND1, C1), jnp.float32),
    )(agg1, deg1, xs, W_neigh1)
    return out


# speedup vs baseline: 1.0600x; 1.0578x over previous
import functools

import jax
import jax.numpy as jnp
from jax import lax
from jax.experimental import pallas as pl
from jax.experimental.pallas import tpu as pltpu
from jax.experimental.pallas import tpu_sc as plsc

N = 10000
D0 = 128
H0 = 128
C1 = 64
ND0 = 5000
ND1 = 1000
E0 = 320000
E1 = 64000

NC = 2
NS = 16
NW = NC * NS
L = 16

PAD0 = 5120
PAD1 = 1024


def _make_sc_agg(EP, D, PAD, CH):
    NB = 4
    EW = EP // NW
    CHUNKS = EW // CH
    GROUPS = CHUNKS // NB
    TAIL = CHUNKS % NB
    RPT = PAD // NS
    mesh = plsc.VectorSubcoreMesh(core_axis_name="c", subcore_axis_name="s")

    @functools.partial(
        pl.kernel,
        mesh=mesh,
        out_type=(
            jax.ShapeDtypeStruct((NC, PAD, D), jnp.float32),
            jax.ShapeDtypeStruct((NW, PAD), jnp.float32),
        ),
        scratch_types=[
            [pltpu.VMEM((CH,), jnp.int32)] * NB,
            [pltpu.VMEM((CH,), jnp.int32)] * NB,
            [pltpu.VMEM((CH, D), jnp.float32)] * NB,
            pltpu.VMEM((PAD,), jnp.float32),
            pltpu.VMEM_SHARED((PAD, D), jnp.float32),
            [pltpu.SemaphoreType.DMA] * NB,
            [pltpu.SemaphoreType.DMA] * NB,
        ],
        compiler_params=pltpu.CompilerParams(needs_layout_passes=False),
    )
    def agg_kernel(table, src_e, dst_e, zacc, zdeg, agg_out, deg_out,
                   src_v, dst_v, rows_v, deg_v, sh_acc, sems, ssems):
        cid = lax.axis_index("c")
        sid = lax.axis_index("s")
        wid = sid * NC + cid
        pltpu.sync_copy(zacc.at[pl.ds(sid * RPT, RPT)],
                        sh_acc.at[pl.ds(sid * RPT, RPT)])
        pltpu.sync_copy(zdeg, deg_v)
        plsc.subcore_barrier()

        base = wid * EW
        ones = jnp.full((L,), 1.0, jnp.float32)

        def start(off, b):
            pltpu.sync_copy(src_e.at[pl.ds(off, CH)], src_v[b])
            pltpu.sync_copy(dst_e.at[pl.ds(off, CH)], dst_v[b])
            return pltpu.async_copy(table.at[src_v[b]], rows_v[b], sems[b])

        def finish(handle, b):
            handle.wait()
            h = pltpu.async_copy(rows_v[b], sh_acc.at[dst_v[b]], ssems[b],
                                 add=True)
            for j in range(CH // L):
                dv = dst_v[b][pl.ds(j * L, L)]
                plsc.addupdate_scatter(deg_v, [dv], ones)
            return h

        def outer(o, carry):
            off = base + o * (NB * CH)
            handles = [start(off + b * CH, b) for b in range(NB)]
            shandles = [finish(handles[b], b) for b in range(NB)]
            for sh in shandles:
                sh.wait()
            return carry

        lax.fori_loop(0, GROUPS, outer, 0)
        for t in range(TAIL):
            off = base + (GROUPS * NB + t) * CH
            finish(start(off, t), t).wait()
        plsc.subcore_barrier()
        pltpu.sync_copy(sh_acc.at[pl.ds(sid * RPT, RPT)],
                        agg_out.at[cid, pl.ds(sid * RPT, RPT)])
        pltpu.sync_copy(deg_v, deg_out.at[wid])

    return agg_kernel


CH = 80
E0P = E0
E1P = E1

_sc_agg0 = _make_sc_agg(E0P, D0, PAD0, CH)
_sc_agg1 = _make_sc_agg(E1P, H0, PAD1, CH)


def _pad_edges(ei, ep, nd, pad):
    npad = ep - ei.shape[1]
    src = jnp.concatenate([ei[0], jnp.zeros((npad,), jnp.int32)])
    cyc = nd + jnp.arange(npad, dtype=jnp.int32) % jnp.int32(pad - nd)
    dst = jnp.concatenate([ei[1], cyc])
    return src, dst


def _tc_layer0_body(agg_ref, deg_ref, h_ref, wn0, ws0, b0r, g0r, be0r,
                    ws1, b1r, x_ref, xs_ref):
    deg = jnp.sum(deg_ref[...], axis=0)[:ND0]
    aggs = agg_ref[0, :ND0, :] + agg_ref[1, :ND0, :]
    mean = aggs / jnp.maximum(deg, 1.0)[:, None]
    dn = (((1,), (1,)), ((), ()))
    z = (lax.dot_general(mean, wn0[...], dn, preferred_element_type=jnp.float32)
         + lax.dot_general(h_ref[...], ws0[...], dn, preferred_element_type=jnp.float32)
         + b0r[...][None, :])
    mu = jnp.mean(z, axis=-1, keepdims=True)
    zc = z - mu
    var = jnp.mean(zc * zc, axis=-1, keepdims=True)
    xn = zc * lax.rsqrt(var + 1e-5)
    x = jnp.maximum(xn * g0r[...][None, :] + be0r[...][None, :], 0.0)
    x_ref[...] = x
    x1 = x[:ND1]
    xs_ref[...] = (lax.dot_general(x1, ws1[...], dn, preferred_element_type=jnp.float32)
                   + b1r[...][None, :])


def _tc_layer1_body(agg_ref, deg_ref, xs_ref, wn1, out_ref):
    deg = jnp.sum(deg_ref[...], axis=0)[:ND1]
    aggs = agg_ref[0, :ND1, :] + agg_ref[1, :ND1, :]
    mean = aggs / jnp.maximum(deg, 1.0)[:, None]
    dn = (((1,), (1,)), ((), ()))
    out_ref[...] = (lax.dot_general(mean, wn1[...], dn,
                                    preferred_element_type=jnp.float32)
                    + xs_ref[...])


def kernel(h, edge_index0, edge_index1, W_neigh0, W_self0, b0,
           gamma0, beta0, W_neigh1, W_self1, b1):
    zacc0 = jnp.zeros((PAD0, D0), jnp.float32)
    zdeg0 = jnp.zeros((PAD0,), jnp.float32)
    zacc1 = jnp.zeros((PAD1, H0), jnp.float32)
    zdeg1 = jnp.zeros((PAD1,), jnp.float32)

    src0, dst0 = _pad_edges(edge_index0, E0P, ND0, PAD0)
    agg0, deg0 = _sc_agg0(h, src0, dst0, zacc0, zdeg0)

    h5k = lax.slice(h, (0, 0), (ND0, D0))
    x, xs = pl.pallas_call(
        _tc_layer0_body,
        out_shape=(
            jax.ShapeDtypeStruct((ND0, H0), jnp.float32),
            jax.ShapeDtypeStruct((ND1, C1), jnp.float32),
        ),
    )(agg0, deg0, h5k, W_neigh0, W_self0, b0, gamma0, beta0,
      W_self1, b1)

    src1, dst1 = _pad_edges(edge_index1, E1P, ND1, PAD1)
    agg1, deg1 = _sc_agg1(x, src1, dst1, zacc1, zdeg1)

    out = pl.pallas_call(
        _tc_layer1_body,
        out_shape=jax.ShapeDtypeStruct((ND1, C1), jnp.float32),
    )(agg1, deg1, xs, W_neigh1)
    return out


# speedup vs baseline: 1.0609x; 1.0009x over previous
import functools

import jax
import jax.numpy as jnp
from jax import lax
from jax.experimental import pallas as pl
from jax.experimental.pallas import tpu as pltpu
from jax.experimental.pallas import tpu_sc as plsc

N = 10000
D0 = 128
H0 = 128
C1 = 64
ND0 = 5000
ND1 = 1000
E0 = 320000
E1 = 64000

NC = 2
NS = 16
NW = NC * NS
L = 16

PAD0 = 5120
PAD1 = 1024


def _make_sc_agg(EP, D, PAD, CH):
    NB = 4
    EW = EP // NW
    CHUNKS = EW // CH
    GROUPS = CHUNKS // NB
    TAIL = CHUNKS % NB
    RPT = PAD // NS
    mesh = plsc.VectorSubcoreMesh(core_axis_name="c", subcore_axis_name="s")

    @functools.partial(
        pl.kernel,
        mesh=mesh,
        out_type=(
            jax.ShapeDtypeStruct((NC, PAD, D), jnp.float32),
            jax.ShapeDtypeStruct((NW, PAD), jnp.float32),
        ),
        scratch_types=[
            [pltpu.VMEM((CH,), jnp.int32)] * NB,
            [pltpu.VMEM((CH,), jnp.int32)] * NB,
            [pltpu.VMEM((CH, D), jnp.float32)] * NB,
            pltpu.VMEM((PAD,), jnp.float32),
            pltpu.VMEM_SHARED((PAD, D), jnp.float32),
            [pltpu.SemaphoreType.DMA] * NB,
            [pltpu.SemaphoreType.DMA] * NB,
        ],
        compiler_params=pltpu.CompilerParams(needs_layout_passes=False),
    )
    def agg_kernel(table, src_e, dst_e, zacc, zdeg, agg_out, deg_out,
                   src_v, dst_v, rows_v, deg_v, sh_acc, sems, ssems):
        cid = lax.axis_index("c")
        sid = lax.axis_index("s")
        wid = sid * NC + cid
        pltpu.sync_copy(zacc.at[pl.ds(sid * RPT, RPT)],
                        sh_acc.at[pl.ds(sid * RPT, RPT)])
        pltpu.sync_copy(zdeg, deg_v)
        plsc.subcore_barrier()

        base = wid * EW
        ones = jnp.full((L,), 1.0, jnp.float32)

        def start(off, b):
            pltpu.sync_copy(src_e.at[pl.ds(off, CH)], src_v[b])
            pltpu.sync_copy(dst_e.at[pl.ds(off, CH)], dst_v[b])
            return pltpu.async_copy(table.at[src_v[b]], rows_v[b], sems[b])

        def finish(handle, b):
            handle.wait()
            h = pltpu.async_copy(rows_v[b], sh_acc.at[dst_v[b]], ssems[b],
                                 add=True)
            for j in range(CH // L):
                dv = dst_v[b][pl.ds(j * L, L)]
                plsc.addupdate_scatter(deg_v, [dv], ones)
            return h

        def outer(o, carry):
            off = base + o * (NB * CH)
            handles = [start(off + b * CH, b) for b in range(NB)]
            shandles = [finish(handles[b], b) for b in range(NB)]
            for sh in shandles:
                sh.wait()
            return carry

        lax.fori_loop(0, GROUPS, outer, 0)
        for t in range(TAIL):
            off = base + (GROUPS * NB + t) * CH
            finish(start(off, t), t).wait()
        plsc.subcore_barrier()
        pltpu.sync_copy(sh_acc.at[pl.ds(sid * RPT, RPT)],
                        agg_out.at[cid, pl.ds(sid * RPT, RPT)])
        pltpu.sync_copy(deg_v, deg_out.at[wid])

    return agg_kernel


CH = 80
E0P = E0
E1P = E1

_sc_agg0 = _make_sc_agg(E0P, D0, PAD0, CH)
_sc_agg1 = _make_sc_agg(E1P, H0, PAD1, CH)


def _tc_layer0_body(agg_ref, deg_ref, h_ref, wn0, ws0, b0r, g0r, be0r,
                    ws1, b1r, x_ref, xs_ref):
    deg = jnp.sum(deg_ref[...], axis=0)[:ND0]
    aggs = agg_ref[0, :ND0, :] + agg_ref[1, :ND0, :]
    mean = aggs / jnp.maximum(deg, 1.0)[:, None]
    dn = (((1,), (1,)), ((), ()))
    z = (lax.dot_general(mean, wn0[...], dn, preferred_element_type=jnp.float32)
         + lax.dot_general(h_ref[...], ws0[...], dn, preferred_element_type=jnp.float32)
         + b0r[...][None, :])
    mu = jnp.mean(z, axis=-1, keepdims=True)
    zc = z - mu
    var = jnp.mean(zc * zc, axis=-1, keepdims=True)
    xn = zc * lax.rsqrt(var + 1e-5)
    x = jnp.maximum(xn * g0r[...][None, :] + be0r[...][None, :], 0.0)
    x_ref[...] = x
    x1 = x[:ND1]
    xs_ref[...] = (lax.dot_general(x1, ws1[...], dn, preferred_element_type=jnp.float32)
                   + b1r[...][None, :])


def _tc_layer1_body(agg_ref, deg_ref, xs_ref, wn1, out_ref):
    deg = jnp.sum(deg_ref[...], axis=0)[:ND1]
    aggs = agg_ref[0, :ND1, :] + agg_ref[1, :ND1, :]
    mean = aggs / jnp.maximum(deg, 1.0)[:, None]
    dn = (((1,), (1,)), ((), ()))
    out_ref[...] = (lax.dot_general(mean, wn1[...], dn,
                                    preferred_element_type=jnp.float32)
                    + xs_ref[...])


def kernel(h, edge_index0, edge_index1, W_neigh0, W_self0, b0,
           gamma0, beta0, W_neigh1, W_self1, b1):
    zacc0 = jnp.zeros((PAD0, D0), jnp.float32)
    zdeg0 = jnp.zeros((PAD0,), jnp.float32)
    zacc1 = jnp.zeros((PAD1, H0), jnp.float32)
    zdeg1 = jnp.zeros((PAD1,), jnp.float32)

    agg0, deg0 = _sc_agg0(h, edge_index0[0], edge_index0[1], zacc0, zdeg0)

    h5k = lax.slice(h, (0, 0), (ND0, D0))
    x, xs = pl.pallas_call(
        _tc_layer0_body,
        out_shape=(
            jax.ShapeDtypeStruct((ND0, H0), jnp.float32),
            jax.ShapeDtypeStruct((ND1, C1), jnp.float32),
        ),
    )(agg0, deg0, h5k, W_neigh0, W_self0, b0, gamma0, beta0,
      W_self1, b1)

    agg1, deg1 = _sc_agg1(x, edge_index1[0], edge_index1[1], zacc1, zdeg1)

    out = pl.pallas_call(
        _tc_layer1_body,
        out_shape=jax.ShapeDtypeStruct((ND1, C1), jnp.float32),
    )(agg1, deg1, xs, W_neigh1)
    return out


# speedup vs baseline: 1.0707x; 1.0092x over previous
import functools

import jax
import jax.numpy as jnp
from jax import lax
from jax.experimental import pallas as pl
from jax.experimental.pallas import tpu as pltpu
from jax.experimental.pallas import tpu_sc as plsc

N = 10000
D0 = 128
H0 = 128
C1 = 64
ND0 = 5000
ND1 = 1000
E0 = 320000
E1 = 64000

NC = 2
NS = 16
NW = NC * NS
L = 16

PAD0 = 5120
PAD1 = 1024


def _make_sc_agg(EP, D, PAD, CH):
    NB = 5
    EW = EP // NW
    CHUNKS = EW // CH
    GROUPS = CHUNKS // NB
    TAIL = CHUNKS % NB
    RPT = PAD // NS
    mesh = plsc.VectorSubcoreMesh(core_axis_name="c", subcore_axis_name="s")

    @functools.partial(
        pl.kernel,
        mesh=mesh,
        out_type=(
            jax.ShapeDtypeStruct((NC, PAD, D), jnp.float32),
            jax.ShapeDtypeStruct((NW, PAD), jnp.float32),
        ),
        scratch_types=[
            [pltpu.VMEM((CH,), jnp.int32)] * NB,
            [pltpu.VMEM((CH,), jnp.int32)] * NB,
            [pltpu.VMEM((CH, D), jnp.float32)] * NB,
            pltpu.VMEM((PAD,), jnp.float32),
            pltpu.VMEM_SHARED((PAD, D), jnp.float32),
            [pltpu.SemaphoreType.DMA] * NB,
            [pltpu.SemaphoreType.DMA] * NB,
        ],
        compiler_params=pltpu.CompilerParams(needs_layout_passes=False),
    )
    def agg_kernel(table, src_e, dst_e, zacc, zdeg, agg_out, deg_out,
                   src_v, dst_v, rows_v, deg_v, sh_acc, sems, ssems):
        cid = lax.axis_index("c")
        sid = lax.axis_index("s")
        wid = sid * NC + cid
        pltpu.sync_copy(zacc.at[pl.ds(sid * RPT, RPT)],
                        sh_acc.at[pl.ds(sid * RPT, RPT)])
        pltpu.sync_copy(zdeg, deg_v)
        plsc.subcore_barrier()

        base = wid * EW
        ones = jnp.full((L,), 1.0, jnp.float32)

        def start(off, b):
            pltpu.sync_copy(src_e.at[pl.ds(off, CH)], src_v[b])
            pltpu.sync_copy(dst_e.at[pl.ds(off, CH)], dst_v[b])
            return pltpu.async_copy(table.at[src_v[b]], rows_v[b], sems[b])

        def finish(handle, b):
            handle.wait()
            h = pltpu.async_copy(rows_v[b], sh_acc.at[dst_v[b]], ssems[b],
                                 add=True)
            for j in range(CH // L):
                dv = dst_v[b][pl.ds(j * L, L)]
                plsc.addupdate_scatter(deg_v, [dv], ones)
            return h

        def outer(o, carry):
            off = base + o * (NB * CH)
            handles = [start(off + b * CH, b) for b in range(NB)]
            shandles = [finish(handles[b], b) for b in range(NB)]
            for sh in shandles:
                sh.wait()
            return carry

        lax.fori_loop(0, GROUPS, outer, 0)
        for t in range(TAIL):
            off = base + (GROUPS * NB + t) * CH
            finish(start(off, t), t).wait()
        plsc.subcore_barrier()
        pltpu.sync_copy(sh_acc.at[pl.ds(sid * RPT, RPT)],
                        agg_out.at[cid, pl.ds(sid * RPT, RPT)])
        pltpu.sync_copy(deg_v, deg_out.at[wid])

    return agg_kernel


CH = 80
E0P = E0
E1P = E1

_sc_agg0 = _make_sc_agg(E0P, D0, PAD0, CH)
_sc_agg1 = _make_sc_agg(E1P, H0, PAD1, CH)


def _tc_layer0_body(agg_ref, deg_ref, h_ref, wn0, ws0, b0r, g0r, be0r,
                    ws1, b1r, x_ref, xs_ref):
    deg = jnp.sum(deg_ref[...], axis=0)[:ND0]
    aggs = agg_ref[0, :ND0, :] + agg_ref[1, :ND0, :]
    mean = aggs / jnp.maximum(deg, 1.0)[:, None]
    dn = (((1,), (1,)), ((), ()))
    z = (lax.dot_general(mean, wn0[...], dn, preferred_element_type=jnp.float32)
         + lax.dot_general(h_ref[...], ws0[...], dn, preferred_element_type=jnp.float32)
         + b0r[...][None, :])
    mu = jnp.mean(z, axis=-1, keepdims=True)
    zc = z - mu
    var = jnp.mean(zc * zc, axis=-1, keepdims=True)
    xn = zc * lax.rsqrt(var + 1e-5)
    x = jnp.maximum(xn * g0r[...][None, :] + be0r[...][None, :], 0.0)
    x_ref[...] = x
    x1 = x[:ND1]
    xs_ref[...] = (lax.dot_general(x1, ws1[...], dn, preferred_element_type=jnp.float32)
                   + b1r[...][None, :])


def _tc_layer1_body(agg_ref, deg_ref, xs_ref, wn1, out_ref):
    deg = jnp.sum(deg_ref[...], axis=0)[:ND1]
    aggs = agg_ref[0, :ND1, :] + agg_ref[1, :ND1, :]
    mean = aggs / jnp.maximum(deg, 1.0)[:, None]
    dn = (((1,), (1,)), ((), ()))
    out_ref[...] = (lax.dot_general(mean, wn1[...], dn,
                                    preferred_element_type=jnp.float32)
                    + xs_ref[...])


def kernel(h, edge_index0, edge_index1, W_neigh0, W_self0, b0,
           gamma0, beta0, W_neigh1, W_self1, b1):
    zacc0 = jnp.zeros((PAD0, D0), jnp.float32)
    zdeg0 = jnp.zeros((PAD0,), jnp.float32)
    zacc1 = jnp.zeros((PAD1, H0), jnp.float32)
    zdeg1 = jnp.zeros((PAD1,), jnp.float32)

    agg0, deg0 = _sc_agg0(h, edge_index0[0], edge_index0[1], zacc0, zdeg0)

    h5k = lax.slice(h, (0, 0), (ND0, D0))
    x, xs = pl.pallas_call(
        _tc_layer0_body,
        out_shape=(
            jax.ShapeDtypeStruct((ND0, H0), jnp.float32),
            jax.ShapeDtypeStruct((ND1, C1), jnp.float32),
        ),
    )(agg0, deg0, h5k, W_neigh0, W_self0, b0, gamma0, beta0,
      W_self1, b1)

    agg1, deg1 = _sc_agg1(x, edge_index1[0], edge_index1[1], zacc1, zdeg1)

    out = pl.pallas_call(
        _tc_layer1_body,
        out_shape=jax.ShapeDtypeStruct((ND1, C1), jnp.float32),
    )(agg1, deg1, xs, W_neigh1)
    return out
